# baseline (speedup 1.0000x reference)
import jax
import jax.numpy as jnp
from jax.experimental import pallas as pl


def _mm_body(a_ref, b_ref, o_ref):
    o_ref[...] = jnp.dot(a_ref[...], b_ref[...], preferred_element_type=jnp.float32)


def _mm(a, b):
    M, K = a.shape
    _, N = b.shape
    BM = 1000
    return pl.pallas_call(
        _mm_body,
        grid=(M // BM,),
        in_specs=[
            pl.BlockSpec((BM, K), lambda i: (i, 0)),
            pl.BlockSpec((K, N), lambda i: (0, 0)),
        ],
        out_specs=pl.BlockSpec((BM, N), lambda i: (i, 0)),
        out_shape=jax.ShapeDtypeStruct((M, N), jnp.float32),
    )(a, b)


def kernel(x, edge_index, edge_attr, W_n1, b_n1, W_e1, b_e1, W_x, W_h, W_e, b_l, W_last, b_last, gamma, beta):
    perm = jnp.argsort(edge_index[1], stable=True)
    src = jnp.take(edge_index[0], perm)
    dst = jnp.take(edge_index[1], perm)
    edge_attr = jnp.take(edge_attr, perm, axis=0)
    n = x.shape[0]
    L = W_x.shape[0]
    edge_agg = jax.ops.segment_sum(edge_attr, dst, num_segments=n,
                                   indices_are_sorted=True)
    h = jax.nn.relu(_mm(x, W_n1) + b_n1 + _mm(edge_agg, W_e1) + b_e1)
    for i in range(L):
        nbr = jax.ops.segment_sum(jnp.take(h, src, axis=0), dst, num_segments=n,
                                  indices_are_sorted=True)
        h = jax.nn.relu(_mm(x, W_x[i]) + _mm(nbr, W_h[i]) + _mm(edge_agg, W_e[i]) + b_l[i])
    z = h @ W_last + b_last
    mu = jnp.mean(z, axis=0)
    var = jnp.var(z, axis=0)
    z = (z - mu) / jnp.sqrt(var + 1e-5) * gamma + beta
    out = jnp.mean(z, axis=0, keepdims=True)
    return out


# speedup vs baseline: 1.5716x; 1.5716x over previous
import functools

import jax
import jax.numpy as jnp
from jax import lax
from jax.experimental import pallas as pl
from jax.experimental.pallas import tpu as pltpu
from jax.experimental.pallas import tpu_sc as plsc

N_EDGES_TOTAL = 320000
N_WORKERS = 32
EDGES_PER_WORKER = N_EDGES_TOTAL // N_WORKERS
CHUNK = 400
IDX_ROWS = 4
IDX_COLS = CHUNK // IDX_ROWS
N_CHUNKS = EDGES_PER_WORKER // CHUNK


def _mm_body(a_ref, b_ref, o_ref):
    o_ref[...] = jnp.dot(a_ref[...], b_ref[...], preferred_element_type=jnp.float32)


def _mm(a, b):
    M, K = a.shape
    _, N = b.shape
    BM = 1000
    return pl.pallas_call(
        _mm_body,
        grid=(M // BM,),
        in_specs=[
            pl.BlockSpec((BM, K), lambda i: (i, 0)),
            pl.BlockSpec((K, N), lambda i: (0, 0)),
        ],
        out_specs=pl.BlockSpec((BM, N), lambda i: (i, 0)),
        out_shape=jax.ShapeDtypeStruct((M, N), jnp.float32),
    )(a, b)


def _sc_gather(h, src2d):
    mesh = plsc.VectorSubcoreMesh(core_axis_name="c", subcore_axis_name="s")

    @functools.partial(
        pl.kernel,
        mesh=mesh,
        out_type=jax.ShapeDtypeStruct((N_EDGES_TOTAL, 128), jnp.float32),
        scratch_types=[
            pltpu.VMEM((IDX_ROWS, IDX_COLS), jnp.int32),
            pltpu.VMEM((CHUNK, 128), jnp.float32),
            pltpu.SemaphoreType.DMA,
        ],
    )
    def k(h_hbm, src_hbm, out_hbm, idx_v, rows_v, sem):
        wid = lax.axis_index("s") * 2 + lax.axis_index("c")
        edge_base = wid * EDGES_PER_WORKER
        row_base = wid * (EDGES_PER_WORKER // IDX_COLS)

        def step(c, carry):
            pltpu.sync_copy(src_hbm.at[pl.ds(row_base + c * IDX_ROWS, IDX_ROWS)],
                            idx_v)
            copies = [
                pltpu.async_copy(h_hbm.at[idx_v.at[j]],
                                 rows_v.at[pl.ds(j * IDX_COLS, IDX_COLS)], sem)
                for j in range(IDX_ROWS)
            ]
            for cp in copies:
                cp.wait()
            pltpu.sync_copy(rows_v,
                            out_hbm.at[pl.ds(edge_base + c * CHUNK, CHUNK)])
            return carry

        lax.fori_loop(0, N_CHUNKS, step, 0)

    return k(h, src2d)


def kernel(x, edge_index, edge_attr, W_n1, b_n1, W_e1, b_e1, W_x, W_h, W_e, b_l, W_last, b_last, gamma, beta):
    src = edge_index[0]
    dst = edge_index[1]
    n = x.shape[0]
    L = W_x.shape[0]
    src2d = src.reshape(N_EDGES_TOTAL // IDX_COLS, IDX_COLS)
    edge_agg = jax.ops.segment_sum(edge_attr, dst, num_segments=n)
    h = jax.nn.relu(_mm(x, W_n1) + b_n1 + _mm(edge_agg, W_e1) + b_e1)
    for i in range(L):
        nbr = jax.ops.segment_sum(_sc_gather(h, src2d), dst, num_segments=n)
        h = jax.nn.relu(_mm(x, W_x[i]) + _mm(nbr, W_h[i]) + _mm(edge_agg, W_e[i]) + b_l[i])
    z = h @ W_last + b_last
    mu = jnp.mean(z, axis=0)
    var = jnp.var(z, axis=0)
    z = (z - mu) / jnp.sqrt(var + 1e-5) * gamma + beta
    out = jnp.mean(z, axis=0, keepdims=True)
    return out


# speedup vs baseline: 1.5735x; 1.0012x over previous
import functools

import jax
import jax.numpy as jnp
from jax import lax
from jax.experimental import pallas as pl
from jax.experimental.pallas import tpu as pltpu
from jax.experimental.pallas import tpu_sc as plsc

N_EDGES_TOTAL = 320000
N_WORKERS = 32
EDGES_PER_WORKER = N_EDGES_TOTAL // N_WORKERS
CHUNK = 400
IDX_ROWS = 4
IDX_COLS = CHUNK // IDX_ROWS
N_CHUNKS = EDGES_PER_WORKER // CHUNK


def _dot(a, b):
    return jnp.dot(a, b, preferred_element_type=jnp.float32)


def _first_body(x_ref, ea_ref, wn_ref, bn_ref, we_ref, be_ref, o_ref):
    o_ref[...] = jnp.maximum(
        ((_dot(x_ref[...], wn_ref[...]) + bn_ref[...])
         + _dot(ea_ref[...], we_ref[...])) + be_ref[...], 0.0)


def _layer_body(x_ref, nbr_ref, ea_ref, wx_ref, wh_ref, we_ref, bl_ref, o_ref):
    o_ref[...] = jnp.maximum(
        ((_dot(x_ref[...], wx_ref[...]) + _dot(nbr_ref[...], wh_ref[...]))
         + _dot(ea_ref[...], we_ref[...])) + bl_ref[...], 0.0)


_BM = 1000


def _row_spec(cols):
    return pl.BlockSpec((_BM, cols), lambda i: (i, 0))


def _full_spec(rows, cols):
    return pl.BlockSpec((rows, cols), lambda i: (0, 0))


def _first_layer(x, ea, wn, bn, we, be):
    return pl.pallas_call(
        _first_body,
        grid=(x.shape[0] // _BM,),
        in_specs=[_row_spec(128), _row_spec(16), _full_spec(128, 128),
                  _full_spec(1, 128), _full_spec(16, 128), _full_spec(1, 128)],
        out_specs=_row_spec(128),
        out_shape=jax.ShapeDtypeStruct((x.shape[0], 128), jnp.float32),
    )(x, ea, wn, bn.reshape(1, 128), we, be.reshape(1, 128))


def _layer(x, nbr, ea, wx, wh, we, bl):
    return pl.pallas_call(
        _layer_body,
        grid=(x.shape[0] // _BM,),
        in_specs=[_row_spec(128), _row_spec(128), _row_spec(16),
                  _full_spec(128, 128), _full_spec(128, 128),
                  _full_spec(16, 128), _full_spec(1, 128)],
        out_specs=_row_spec(128),
        out_shape=jax.ShapeDtypeStruct((x.shape[0], 128), jnp.float32),
    )(x, nbr, ea, wx, wh, we, bl.reshape(1, 128))


def _sc_gather(h, src2d):
    mesh = plsc.VectorSubcoreMesh(core_axis_name="c", subcore_axis_name="s")

    @functools.partial(
        pl.kernel,
        mesh=mesh,
        out_type=jax.ShapeDtypeStruct((N_EDGES_TOTAL, 128), jnp.float32),
        scratch_types=[
            pltpu.VMEM((IDX_ROWS, IDX_COLS), jnp.int32),
            pltpu.VMEM((CHUNK, 128), jnp.float32),
            pltpu.SemaphoreType.DMA,
        ],
    )
    def k(h_hbm, src_hbm, out_hbm, idx_v, rows_v, sem):
        wid = lax.axis_index("s") * 2 + lax.axis_index("c")
        edge_base = wid * EDGES_PER_WORKER
        row_base = wid * (EDGES_PER_WORKER // IDX_COLS)

        def step(c, carry):
            pltpu.sync_copy(src_hbm.at[pl.ds(row_base + c * IDX_ROWS, IDX_ROWS)],
                            idx_v)
            copies = [
                pltpu.async_copy(h_hbm.at[idx_v.at[j]],
                                 rows_v.at[pl.ds(j * IDX_COLS, IDX_COLS)], sem)
                for j in range(IDX_ROWS)
            ]
            for cp in copies:
                cp.wait()
            pltpu.sync_copy(rows_v,
                            out_hbm.at[pl.ds(edge_base + c * CHUNK, CHUNK)])
            return carry

        lax.fori_loop(0, N_CHUNKS, step, 0)

    return k(h, src2d)


def kernel(x, edge_index, edge_attr, W_n1, b_n1, W_e1, b_e1, W_x, W_h, W_e, b_l, W_last, b_last, gamma, beta):
    src = edge_index[0]
    dst = edge_index[1]
    n = x.shape[0]
    L = W_x.shape[0]
    src2d = src.reshape(N_EDGES_TOTAL // IDX_COLS, IDX_COLS)
    edge_agg = jax.ops.segment_sum(edge_attr, dst, num_segments=n)
    h = _first_layer(x, edge_agg, W_n1, b_n1, W_e1, b_e1)
    for i in range(L):
        nbr = jax.ops.segment_sum(_sc_gather(h, src2d), dst, num_segments=n)
        h = _layer(x, nbr, edge_agg, W_x[i], W_h[i], W_e[i], b_l[i])
    z = h @ W_last + b_last
    mu = jnp.mean(z, axis=0)
    var = jnp.var(z, axis=0)
    z = (z - mu) / jnp.sqrt(var + 1e-5) * gamma + beta
    out = jnp.mean(z, axis=0, keepdims=True)
    return out
